# baseline (speedup 1.0000x reference)
import functools

import jax
import jax.numpy as jnp
from jax import lax
from jax.experimental import pallas as pl
from jax.experimental.pallas import tpu as pltpu
from jax.experimental.pallas import tpu_sc as plsc

B, S = 8, 576
N = B * S
IN_DIM, HID, K_DIM, Z_DIM = 768, 2048, 1024, 64
M_BLK = 128
M_DEC = 512
KC = 128
NKC = K_DIM // KC

NC, NS = 2, 16
NW = NC * NS
B_TOT = 2 * N
B_PER_W = B_TOT // NW
GC = 96
NGC = B_PER_W // GC

F32 = jnp.float32
_DEF = jax.lax.Precision.DEFAULT
_HI = jax.lax.Precision.HIGHEST


def _enc_block(x_ref, w1_ref, b1_ref, w2_ref, b2_ref, et_ref,
               zenc_ref, idx1_ref, idx2_ref):
    h = jnp.maximum(jnp.dot(x_ref[...], w1_ref[...], precision=_DEF)
                    + b1_ref[...], 0.0)
    z = jnp.dot(h, w2_ref[...], precision=_DEF) + b2_ref[...]
    zenc_ref[...] = z

    et = et_ref[...]
    lane = jax.lax.broadcasted_iota(jnp.int32, (M_BLK, KC), 1)
    sc, run_min = [], None
    for c in range(NKC):
        etc = et[:, c * KC:(c + 1) * KC]
        se_c = jnp.sum(etc * etc, axis=0, keepdims=True)
        s = se_c - 2.0 * jnp.dot(z, etc, precision=_HI)
        sc.append(s)
        run_min = s if run_min is None else jnp.minimum(run_min, s)
    gmin = jnp.min(run_min, axis=1, keepdims=True)

    def argmin_from(chunks, gm):
        cand = None
        for c in range(NKC):
            cc = jnp.where(chunks[c] == gm, lane + c * KC, K_DIM)
            cand = cc if cand is None else jnp.minimum(cand, cc)
        return jnp.min(cand, axis=1, keepdims=True)

    idx1 = argmin_from(sc, gmin)

    sc2, run_min2 = [], None
    for c in range(NKC):
        s2 = jnp.where(lane + c * KC == idx1, jnp.inf, sc[c])
        sc2.append(s2)
        run_min2 = s2 if run_min2 is None else jnp.minimum(run_min2, s2)
    gmin2 = jnp.min(run_min2, axis=1, keepdims=True)
    idx2 = argmin_from(sc2, gmin2)

    idx1_ref[...] = idx1
    idx2_ref[...] = idx2


def _sc_gather(table_hbm, idx_hbm, out_hbm,
               i0, i1, i2, r0, r1, r2,
               si0, si1, si2, sg0, sg1, sg2, so0, so1, so2):
    wid = lax.axis_index("s") * NC + lax.axis_index("c")
    base = wid * B_PER_W
    idxs, rows = (i0, i1, i2), (r0, r1, r2)
    sis, sgs, sos = (si0, si1, si2), (sg0, sg1, sg2), (so0, so1, so2)
    icps = [pltpu.async_copy(idx_hbm.at[pl.ds(base + g * GC, GC)],
                             idxs[g], sis[g]) for g in range(NGC)]
    gcps = [None] * NGC
    ocps = [None] * NGC
    for g in range(NGC):
        icps[g].wait()
        gcps[g] = pltpu.async_copy(table_hbm.at[idxs[g]], rows[g], sgs[g])
    for g in range(NGC):
        gcps[g].wait()
        ocps[g] = pltpu.async_copy(rows[g], out_hbm.at[pl.ds(base + g * GC, GC)],
                                   sos[g])
    for g in range(NGC):
        ocps[g].wait()


def _dec_block(z_ref, e1_ref, e2_ref, i1_ref, i2_ref,
               w3_ref, b3_ref, w4_ref, b4_ref, recon_ref, zemb_ref):
    z = z_ref[...]
    e1 = e1_ref[:, :Z_DIM]
    e2 = e2_ref[:, :Z_DIM]
    d1 = jnp.sum((z - e1) ** 2, axis=1, keepdims=True)
    d2 = jnp.sum((z - e2) ** 2, axis=1, keepdims=True)
    swap = (d2 < d1) | ((d2 == d1) & (i2_ref[...] < i1_ref[...]))
    e_sel = jnp.where(swap, e2, e1)
    zemb_ref[...] = e_sel

    h2 = jnp.dot(e_sel, w3_ref[...], precision=_DEF) + b3_ref[...]
    h2 = jnp.where(h2 > 0, h2, 0.1 * h2)
    recon_ref[...] = jnp.dot(h2, w4_ref[...], precision=_DEF) + b4_ref[...]


@jax.jit
def _run(X, W1, b1, W2, b2, embd, W3, b3, W4, b4):
    x2 = X.reshape(N, IN_DIM)
    full = lambda shape: pl.BlockSpec(shape, lambda i: (0, 0))

    zenc, idx1, idx2 = pl.pallas_call(
        _enc_block,
        grid=(N // M_BLK,),
        in_specs=[
            pl.BlockSpec((M_BLK, IN_DIM), lambda i: (i, 0)),
            full((IN_DIM, HID)),
            full((1, HID)),
            full((HID, Z_DIM)),
            full((1, Z_DIM)),
            full((Z_DIM, K_DIM)),
        ],
        out_specs=[
            pl.BlockSpec((M_BLK, Z_DIM), lambda i: (i, 0)),
            pl.BlockSpec((M_BLK, 1), lambda i: (i, 0)),
            pl.BlockSpec((M_BLK, 1), lambda i: (i, 0)),
        ],
        out_shape=[
            jax.ShapeDtypeStruct((N, Z_DIM), F32),
            jax.ShapeDtypeStruct((N, 1), jnp.int32),
            jax.ShapeDtypeStruct((N, 1), jnp.int32),
        ],
    )(x2, W1, b1.reshape(1, HID), W2, b2.reshape(1, Z_DIM), embd.T)

    idx_all = jnp.concatenate([idx1[:, 0], idx2[:, 0]])
    mesh = plsc.VectorSubcoreMesh(core_axis_name="c", subcore_axis_name="s")
    embd_pad = jnp.pad(embd, ((0, 0), (0, 128 - Z_DIM)))
    gathered = pl.kernel(
        _sc_gather,
        mesh=mesh,
        out_type=jax.ShapeDtypeStruct((B_TOT, 128), F32),
        scratch_types=(
            [pltpu.VMEM((GC,), jnp.int32)] * NGC
            + [pltpu.VMEM((GC, 128), F32)] * NGC
            + [pltpu.SemaphoreType.DMA] * (3 * NGC)
        ),
    )(embd_pad, idx_all)
    e1 = gathered[:N]
    e2 = gathered[N:]

    recon, zemb = pl.pallas_call(
        _dec_block,
        grid=(N // M_DEC,),
        in_specs=[
            pl.BlockSpec((M_DEC, Z_DIM), lambda i: (i, 0)),
            pl.BlockSpec((M_DEC, 128), lambda i: (i, 0)),
            pl.BlockSpec((M_DEC, 128), lambda i: (i, 0)),
            pl.BlockSpec((M_DEC, 1), lambda i: (i, 0)),
            pl.BlockSpec((M_DEC, 1), lambda i: (i, 0)),
            full((Z_DIM, HID)),
            full((1, HID)),
            full((HID, IN_DIM)),
            full((1, IN_DIM)),
        ],
        out_specs=[
            pl.BlockSpec((M_DEC, IN_DIM), lambda i: (i, 0)),
            pl.BlockSpec((M_DEC, Z_DIM), lambda i: (i, 0)),
        ],
        out_shape=[
            jax.ShapeDtypeStruct((N, IN_DIM), F32),
            jax.ShapeDtypeStruct((N, Z_DIM), F32),
        ],
    )(zenc, e1, e2, idx1, idx2, W3, b3.reshape(1, HID), W4,
      b4.reshape(1, IN_DIM))

    return (recon.reshape(B, S, IN_DIM), zenc.reshape(B, S, Z_DIM),
            zemb.reshape(B, S, Z_DIM))


def kernel(X, W1, b1, W2, b2, embd, W3, b3, W4, b4):
    return _run(X, W1, b1, W2, b2, embd, W3, b3, W4, b4)


# speedup vs baseline: 1.0903x; 1.0903x over previous
import functools

import jax
import jax.numpy as jnp
from jax import lax
from jax.experimental import pallas as pl
from jax.experimental.pallas import tpu as pltpu
from jax.experimental.pallas import tpu_sc as plsc

B, S = 8, 576
N = B * S
IN_DIM, HID, K_DIM, Z_DIM = 768, 2048, 1024, 64
M_BLK = 512
M_DEC = 512
KC = 128
NKC = K_DIM // KC

NC, NS = 2, 16
NW = NC * NS
B_TOT = 2 * N
B_PER_W = B_TOT // NW
GC = 96
NGC = B_PER_W // GC

F32 = jnp.float32
_DEF = jax.lax.Precision.DEFAULT
_HI = jax.lax.Precision.HIGHEST


def _enc_block(x_ref, w1_ref, b1_ref, w2_ref, b2_ref, et_ref,
               zenc_ref, idx1_ref, idx2_ref):
    h = jnp.maximum(jnp.dot(x_ref[...], w1_ref[...], precision=_DEF)
                    + b1_ref[...], 0.0)
    z = jnp.dot(h, w2_ref[...], precision=_DEF) + b2_ref[...]
    zenc_ref[...] = z

    et = et_ref[...]
    lane = jax.lax.broadcasted_iota(jnp.int32, (M_BLK, KC), 1)
    sc, run_min = [], None
    for c in range(NKC):
        etc = et[:, c * KC:(c + 1) * KC]
        se_c = jnp.sum(etc * etc, axis=0, keepdims=True)
        s = se_c - 2.0 * jnp.dot(z, etc, precision=_HI)
        sc.append(s)
        run_min = s if run_min is None else jnp.minimum(run_min, s)
    gmin = jnp.min(run_min, axis=1, keepdims=True)

    def argmin_from(chunks, gm):
        cand = None
        for c in range(NKC):
            cc = jnp.where(chunks[c] == gm, lane + c * KC, K_DIM)
            cand = cc if cand is None else jnp.minimum(cand, cc)
        return jnp.min(cand, axis=1, keepdims=True)

    idx1 = argmin_from(sc, gmin)

    sc2, run_min2 = [], None
    for c in range(NKC):
        s2 = jnp.where(lane + c * KC == idx1, jnp.inf, sc[c])
        sc2.append(s2)
        run_min2 = s2 if run_min2 is None else jnp.minimum(run_min2, s2)
    gmin2 = jnp.min(run_min2, axis=1, keepdims=True)
    idx2 = argmin_from(sc2, gmin2)

    idx1_ref[...] = idx1
    idx2_ref[...] = idx2


def _sc_gather(table_hbm, idx_hbm, out_hbm,
               i0, i1, i2, r0, r1, r2,
               si0, si1, si2, sg0, sg1, sg2, so0, so1, so2):
    wid = lax.axis_index("s") * NC + lax.axis_index("c")
    base = wid * B_PER_W
    idxs, rows = (i0, i1, i2), (r0, r1, r2)
    sis, sgs, sos = (si0, si1, si2), (sg0, sg1, sg2), (so0, so1, so2)
    icps = [pltpu.async_copy(idx_hbm.at[pl.ds(base + g * GC, GC)],
                             idxs[g], sis[g]) for g in range(NGC)]
    gcps = [None] * NGC
    ocps = [None] * NGC
    for g in range(NGC):
        icps[g].wait()
        gcps[g] = pltpu.async_copy(table_hbm.at[idxs[g]], rows[g], sgs[g])
    for g in range(NGC):
        gcps[g].wait()
        ocps[g] = pltpu.async_copy(rows[g], out_hbm.at[pl.ds(base + g * GC, GC)],
                                   sos[g])
    for g in range(NGC):
        ocps[g].wait()


def _dec_block(z_ref, e1_ref, e2_ref, i1_ref, i2_ref,
               w3_ref, b3_ref, w4_ref, b4_ref, recon_ref, zemb_ref):
    z = z_ref[...]
    e1 = e1_ref[:, :Z_DIM]
    e2 = e2_ref[:, :Z_DIM]
    d1 = jnp.sum((z - e1) ** 2, axis=1, keepdims=True)
    d2 = jnp.sum((z - e2) ** 2, axis=1, keepdims=True)
    swap = (d2 < d1) | ((d2 == d1) & (i2_ref[...] < i1_ref[...]))
    e_sel = jnp.where(swap, e2, e1)
    zemb_ref[...] = e_sel

    h2 = jnp.dot(e_sel, w3_ref[...], precision=_DEF) + b3_ref[...]
    h2 = jnp.where(h2 > 0, h2, 0.1 * h2)
    recon_ref[...] = jnp.dot(h2, w4_ref[...], precision=_DEF) + b4_ref[...]


@jax.jit
def _run(X, W1, b1, W2, b2, embd, W3, b3, W4, b4):
    x2 = X.reshape(N, IN_DIM)
    full = lambda shape: pl.BlockSpec(shape, lambda i: (0, 0))

    zenc, idx1, idx2 = pl.pallas_call(
        _enc_block,
        grid=(N // M_BLK,),
        in_specs=[
            pl.BlockSpec((M_BLK, IN_DIM), lambda i: (i, 0)),
            full((IN_DIM, HID)),
            full((1, HID)),
            full((HID, Z_DIM)),
            full((1, Z_DIM)),
            full((Z_DIM, K_DIM)),
        ],
        out_specs=[
            pl.BlockSpec((M_BLK, Z_DIM), lambda i: (i, 0)),
            pl.BlockSpec((M_BLK, 1), lambda i: (i, 0)),
            pl.BlockSpec((M_BLK, 1), lambda i: (i, 0)),
        ],
        out_shape=[
            jax.ShapeDtypeStruct((N, Z_DIM), F32),
            jax.ShapeDtypeStruct((N, 1), jnp.int32),
            jax.ShapeDtypeStruct((N, 1), jnp.int32),
        ],
    )(x2, W1, b1.reshape(1, HID), W2, b2.reshape(1, Z_DIM), embd.T)

    idx_all = jnp.concatenate([idx1[:, 0], idx2[:, 0]])
    mesh = plsc.VectorSubcoreMesh(core_axis_name="c", subcore_axis_name="s")
    embd_pad = jnp.pad(embd, ((0, 0), (0, 128 - Z_DIM)))
    gathered = pl.kernel(
        _sc_gather,
        mesh=mesh,
        out_type=jax.ShapeDtypeStruct((B_TOT, 128), F32),
        scratch_types=(
            [pltpu.VMEM((GC,), jnp.int32)] * NGC
            + [pltpu.VMEM((GC, 128), F32)] * NGC
            + [pltpu.SemaphoreType.DMA] * (3 * NGC)
        ),
    )(embd_pad, idx_all)
    e1 = gathered[:N]
    e2 = gathered[N:]

    recon, zemb = pl.pallas_call(
        _dec_block,
        grid=(N // M_DEC,),
        in_specs=[
            pl.BlockSpec((M_DEC, Z_DIM), lambda i: (i, 0)),
            pl.BlockSpec((M_DEC, 128), lambda i: (i, 0)),
            pl.BlockSpec((M_DEC, 128), lambda i: (i, 0)),
            pl.BlockSpec((M_DEC, 1), lambda i: (i, 0)),
            pl.BlockSpec((M_DEC, 1), lambda i: (i, 0)),
            full((Z_DIM, HID)),
            full((1, HID)),
            full((HID, IN_DIM)),
            full((1, IN_DIM)),
        ],
        out_specs=[
            pl.BlockSpec((M_DEC, IN_DIM), lambda i: (i, 0)),
            pl.BlockSpec((M_DEC, Z_DIM), lambda i: (i, 0)),
        ],
        out_shape=[
            jax.ShapeDtypeStruct((N, IN_DIM), F32),
            jax.ShapeDtypeStruct((N, Z_DIM), F32),
        ],
    )(zenc, e1, e2, idx1, idx2, W3, b3.reshape(1, HID), W4,
      b4.reshape(1, IN_DIM))

    return (recon.reshape(B, S, IN_DIM), zenc.reshape(B, S, Z_DIM),
            zemb.reshape(B, S, Z_DIM))


def kernel(X, W1, b1, W2, b2, embd, W3, b3, W4, b4):
    return _run(X, W1, b1, W2, b2, embd, W3, b3, W4, b4)


# speedup vs baseline: 1.1179x; 1.0253x over previous
import functools

import jax
import jax.numpy as jnp
from jax import lax
from jax.experimental import pallas as pl
from jax.experimental.pallas import tpu as pltpu
from jax.experimental.pallas import tpu_sc as plsc

B, S = 8, 576
N = B * S
IN_DIM, HID, K_DIM, Z_DIM = 768, 2048, 1024, 64
M_BLK = 512
M_DEC = 512
KC = 128
NKC = K_DIM // KC

NC, NS = 2, 16
NW = NC * NS
B_TOT = 2 * N
B_PER_W = N // NW
GC = 72
NGC = B_PER_W // GC

F32 = jnp.float32
_DEF = jax.lax.Precision.DEFAULT
_HI = jax.lax.Precision.HIGHEST


def _enc_block(x_ref, w1_ref, b1_ref, w2_ref, b2_ref, et_ref,
               zenc_ref, idx1_ref, idx2_ref):
    h = jnp.maximum(jnp.dot(x_ref[...], w1_ref[...], precision=_DEF)
                    + b1_ref[...], 0.0)
    z = jnp.dot(h, w2_ref[...], precision=_DEF) + b2_ref[...]
    zenc_ref[...] = z

    et = et_ref[...]
    lane = jax.lax.broadcasted_iota(jnp.int32, (M_BLK, KC), 1)
    sc, run_min = [], None
    for c in range(NKC):
        etc = et[:, c * KC:(c + 1) * KC]
        se_c = jnp.sum(etc * etc, axis=0, keepdims=True)
        s = se_c - 2.0 * jnp.dot(z, etc, precision=_HI)
        sc.append(s)
        run_min = s if run_min is None else jnp.minimum(run_min, s)
    gmin = jnp.min(run_min, axis=1, keepdims=True)

    def argmin_from(chunks, gm):
        cand = None
        for c in range(NKC):
            cc = jnp.where(chunks[c] == gm, lane + c * KC, K_DIM)
            cand = cc if cand is None else jnp.minimum(cand, cc)
        return jnp.min(cand, axis=1, keepdims=True)

    idx1 = argmin_from(sc, gmin)

    sc2, run_min2 = [], None
    for c in range(NKC):
        s2 = jnp.where(lane + c * KC == idx1, jnp.inf, sc[c])
        sc2.append(s2)
        run_min2 = s2 if run_min2 is None else jnp.minimum(run_min2, s2)
    gmin2 = jnp.min(run_min2, axis=1, keepdims=True)
    idx2 = argmin_from(sc2, gmin2)

    idx1_ref[...] = idx1
    idx2_ref[...] = idx2


def _sc_gather(table_hbm, idx1_hbm, idx2_hbm, out_hbm,
               i0, i1, i2, i3, r0, r1, r2, r3,
               si0, si1, si2, si3, sg0, sg1, sg2, sg3,
               so0, so1, so2, so3):
    wid = lax.axis_index("s") * NC + lax.axis_index("c")
    base = wid * B_PER_W
    idxs, rows = (i0, i1, i2, i3), (r0, r1, r2, r3)
    sis, sgs = (si0, si1, si2, si3), (sg0, sg1, sg2, sg3)
    sos = (so0, so1, so2, so3)
    srcs = (idx1_hbm, idx2_hbm)
    offs = []
    for h in range(2):
        for g in range(NGC):
            offs.append((h * NGC + g, srcs[h], base + g * GC, h * N))
    icps = [pltpu.async_copy(src.at[pl.ds(off, GC)], idxs[k], sis[k])
            for k, src, off, _ in offs]
    gcps = [None] * 4
    ocps = [None] * 4
    for k, _, _, _ in offs:
        icps[k].wait()
        gcps[k] = pltpu.async_copy(table_hbm.at[idxs[k]], rows[k], sgs[k])
    for k, _, off, obase in offs:
        gcps[k].wait()
        ocps[k] = pltpu.async_copy(rows[k], out_hbm.at[pl.ds(obase + off, GC)],
                                   sos[k])
    for k, _, _, _ in offs:
        ocps[k].wait()


def _dec_block(z_ref, e1_ref, e2_ref, i1_ref, i2_ref,
               w3_ref, b3_ref, w4_ref, b4_ref, recon_ref, zemb_ref):
    z = z_ref[...]
    e1 = e1_ref[:, :Z_DIM]
    e2 = e2_ref[:, :Z_DIM]
    d1 = jnp.sum((z - e1) ** 2, axis=1, keepdims=True)
    d2 = jnp.sum((z - e2) ** 2, axis=1, keepdims=True)
    swap = (d2 < d1) | ((d2 == d1) & (i2_ref[...] < i1_ref[...]))
    e_sel = jnp.where(swap, e2, e1)
    zemb_ref[...] = e_sel

    h2 = jnp.dot(e_sel, w3_ref[...], precision=_DEF) + b3_ref[...]
    h2 = jnp.where(h2 > 0, h2, 0.1 * h2)
    recon_ref[...] = jnp.dot(h2, w4_ref[...], precision=_DEF) + b4_ref[...]


@jax.jit
def _run(X, W1, b1, W2, b2, embd, W3, b3, W4, b4):
    x2 = X.reshape(N, IN_DIM)
    full = lambda shape: pl.BlockSpec(shape, lambda i: (0, 0))

    zenc, idx1, idx2 = pl.pallas_call(
        _enc_block,
        grid=(N // M_BLK,),
        in_specs=[
            pl.BlockSpec((M_BLK, IN_DIM), lambda i: (i, 0)),
            full((IN_DIM, HID)),
            full((1, HID)),
            full((HID, Z_DIM)),
            full((1, Z_DIM)),
            full((Z_DIM, K_DIM)),
        ],
        out_specs=[
            pl.BlockSpec((M_BLK, Z_DIM), lambda i: (i, 0)),
            pl.BlockSpec((M_BLK, 1), lambda i: (i, 0)),
            pl.BlockSpec((M_BLK, 1), lambda i: (i, 0)),
        ],
        out_shape=[
            jax.ShapeDtypeStruct((N, Z_DIM), F32),
            jax.ShapeDtypeStruct((N, 1), jnp.int32),
            jax.ShapeDtypeStruct((N, 1), jnp.int32),
        ],
    )(x2, W1, b1.reshape(1, HID), W2, b2.reshape(1, Z_DIM), embd.T)

    mesh = plsc.VectorSubcoreMesh(core_axis_name="c", subcore_axis_name="s")
    embd_pad = jnp.pad(embd, ((0, 0), (0, 128 - Z_DIM)))
    gathered = pl.kernel(
        _sc_gather,
        mesh=mesh,
        out_type=jax.ShapeDtypeStruct((B_TOT, 128), F32),
        scratch_types=(
            [pltpu.VMEM((GC,), jnp.int32)] * 4
            + [pltpu.VMEM((GC, 128), F32)] * 4
            + [pltpu.SemaphoreType.DMA] * 12
        ),
    )(embd_pad, idx1.reshape(N), idx2.reshape(N))

    recon, zemb = pl.pallas_call(
        _dec_block,
        grid=(N // M_DEC,),
        in_specs=[
            pl.BlockSpec((M_DEC, Z_DIM), lambda i: (i, 0)),
            pl.BlockSpec((M_DEC, 128), lambda i: (i, 0)),
            pl.BlockSpec((M_DEC, 128), lambda i: (i + N // M_DEC, 0)),
            pl.BlockSpec((M_DEC, 1), lambda i: (i, 0)),
            pl.BlockSpec((M_DEC, 1), lambda i: (i, 0)),
            full((Z_DIM, HID)),
            full((1, HID)),
            full((HID, IN_DIM)),
            full((1, IN_DIM)),
        ],
        out_specs=[
            pl.BlockSpec((M_DEC, IN_DIM), lambda i: (i, 0)),
            pl.BlockSpec((M_DEC, Z_DIM), lambda i: (i, 0)),
        ],
        out_shape=[
            jax.ShapeDtypeStruct((N, IN_DIM), F32),
            jax.ShapeDtypeStruct((N, Z_DIM), F32),
        ],
    )(zenc, gathered, gathered, idx1, idx2, W3, b3.reshape(1, HID), W4,
      b4.reshape(1, IN_DIM))

    return (recon.reshape(B, S, IN_DIM), zenc.reshape(B, S, Z_DIM),
            zemb.reshape(B, S, Z_DIM))


def kernel(X, W1, b1, W2, b2, embd, W3, b3, W4, b4):
    return _run(X, W1, b1, W2, b2, embd, W3, b3, W4, b4)


# speedup vs baseline: 1.1201x; 1.0020x over previous
import jax
import jax.numpy as jnp
from jax import lax
from jax.experimental import pallas as pl
from jax.experimental.pallas import tpu as pltpu
from jax.experimental.pallas import tpu_sc as plsc

B, S = 8, 576
N = B * S
IN_DIM, HID, K_DIM, Z_DIM = 768, 2048, 1024, 64
M_BLK = 512
M_DEC = 512
KC = 128
NKC = K_DIM // KC

NC, NS = 2, 16
NW = NC * NS
B_TOT = 2 * N
B_PER_W = N // NW
GC = 72
NGC = B_PER_W // GC

F32 = jnp.float32
_DEF = jax.lax.Precision.DEFAULT
_HI = jax.lax.Precision.HIGHEST


def _enc_block(x_ref, w1_ref, b1_ref, w2_ref, b2_ref, et_ref,
               zenc_ref, idx1_ref, idx2_ref):
    h = jnp.maximum(jnp.dot(x_ref[...], w1_ref[...], precision=_DEF)
                    + b1_ref[...], 0.0)
    z = jnp.dot(h, w2_ref[...], precision=_DEF) + b2_ref[...]
    zenc_ref[...] = z

    et = et_ref[...]
    lane = jax.lax.broadcasted_iota(jnp.int32, (M_BLK, KC), 1)
    sc, run_min = [], None
    for c in range(NKC):
        etc = et[:, c * KC:(c + 1) * KC]
        se_c = jnp.sum(etc * etc, axis=0, keepdims=True)
        s = se_c - 2.0 * jnp.dot(z, etc, precision=_HI)
        sc.append(s)
        run_min = s if run_min is None else jnp.minimum(run_min, s)
    gmin = jnp.min(run_min, axis=1, keepdims=True)

    def argmin_from(chunks, gm):
        cand = None
        for c in range(NKC):
            cc = jnp.where(chunks[c] == gm, lane + c * KC, K_DIM)
            cand = cc if cand is None else jnp.minimum(cand, cc)
        return jnp.min(cand, axis=1, keepdims=True)

    idx1 = argmin_from(sc, gmin)

    sc2, run_min2 = [], None
    for c in range(NKC):
        s2 = jnp.where(lane + c * KC == idx1, jnp.inf, sc[c])
        sc2.append(s2)
        run_min2 = s2 if run_min2 is None else jnp.minimum(run_min2, s2)
    gmin2 = jnp.min(run_min2, axis=1, keepdims=True)
    idx2 = argmin_from(sc2, gmin2)

    idx1_ref[...] = idx1
    idx2_ref[...] = idx2


def _sc_gather(table_hbm, idx1_hbm, idx2_hbm, out_hbm,
               i0, i1, i2, i3, r0, r1, r2, r3,
               si0, si1, si2, si3, sg0, sg1, sg2, sg3,
               so0, so1, so2, so3):
    wid = lax.axis_index("s") * NC + lax.axis_index("c")
    base = wid * B_PER_W
    idxs, rows = (i0, i1, i2, i3), (r0, r1, r2, r3)
    sis, sgs = (si0, si1, si2, si3), (sg0, sg1, sg2, sg3)
    sos = (so0, so1, so2, so3)
    srcs = (idx1_hbm, idx2_hbm)
    offs = []
    for h in range(2):
        for g in range(NGC):
            offs.append((h * NGC + g, srcs[h], base + g * GC, h * N))
    icps = [pltpu.async_copy(src.at[pl.ds(off, GC)], idxs[k], sis[k])
            for k, src, off, _ in offs]
    gcps = [None] * 4
    ocps = [None] * 4
    for k, _, _, _ in offs:
        icps[k].wait()
        gcps[k] = pltpu.async_copy(table_hbm.at[idxs[k]], rows[k], sgs[k])
    for k, _, off, obase in offs:
        gcps[k].wait()
        ocps[k] = pltpu.async_copy(rows[k], out_hbm.at[pl.ds(obase + off, GC)],
                                   sos[k])
    for k, _, _, _ in offs:
        ocps[k].wait()


def _dec_block(z_ref, e1_ref, e2_ref, i1_ref, i2_ref,
               w3_ref, b3_ref, w4_ref, b4_ref, recon_ref, zemb_ref):
    z = z_ref[...]
    e1 = e1_ref[:, :Z_DIM]
    e2 = e2_ref[:, :Z_DIM]
    d1 = jnp.sum((z - e1) ** 2, axis=1, keepdims=True)
    d2 = jnp.sum((z - e2) ** 2, axis=1, keepdims=True)
    swap = (d2 < d1) | ((d2 == d1) & (i2_ref[...] < i1_ref[...]))
    e_sel = jnp.where(swap, e2, e1)
    zemb_ref[...] = e_sel

    h2 = jnp.dot(e_sel, w3_ref[...], precision=_DEF) + b3_ref[...]
    h2 = jnp.where(h2 > 0, h2, 0.1 * h2)
    recon_ref[...] = jnp.dot(h2, w4_ref[...], precision=_DEF) + b4_ref[...]


@jax.jit
def _run(X, W1, b1, W2, b2, embd, W3, b3, W4, b4):
    x2 = X.reshape(N, IN_DIM)
    full = lambda shape: pl.BlockSpec(shape, lambda i: (0, 0))

    zenc, idx1, idx2 = pl.pallas_call(
        _enc_block,
        grid=(N // M_BLK,),
        in_specs=[
            pl.BlockSpec((M_BLK, IN_DIM), lambda i: (i, 0)),
            full((IN_DIM, HID)),
            full((1, HID)),
            full((HID, Z_DIM)),
            full((1, Z_DIM)),
            full((Z_DIM, K_DIM)),
        ],
        out_specs=[
            pl.BlockSpec((M_BLK, Z_DIM), lambda i: (i, 0)),
            pl.BlockSpec((M_BLK, 1), lambda i: (i, 0)),
            pl.BlockSpec((M_BLK, 1), lambda i: (i, 0)),
        ],
        out_shape=[
            jax.ShapeDtypeStruct((N, Z_DIM), F32),
            jax.ShapeDtypeStruct((N, 1), jnp.int32),
            jax.ShapeDtypeStruct((N, 1), jnp.int32),
        ],
    )(x2, W1, b1.reshape(1, HID), W2, b2.reshape(1, Z_DIM), embd.T)

    mesh = plsc.VectorSubcoreMesh(core_axis_name="c", subcore_axis_name="s")
    embd_pad = jnp.pad(embd, ((0, 0), (0, 128 - Z_DIM)))
    gathered = pl.kernel(
        _sc_gather,
        mesh=mesh,
        out_type=jax.ShapeDtypeStruct((B_TOT, 128), F32),
        scratch_types=(
            [pltpu.VMEM((GC,), jnp.int32)] * 4
            + [pltpu.VMEM((GC, 128), F32)] * 4
            + [pltpu.SemaphoreType.DMA] * 12
        ),
    )(embd_pad, idx1.reshape(N), idx2.reshape(N))

    recon, zemb = pl.pallas_call(
        _dec_block,
        grid=(N // M_DEC,),
        in_specs=[
            pl.BlockSpec((M_DEC, Z_DIM), lambda i: (i, 0)),
            pl.BlockSpec((M_DEC, 128), lambda i: (i, 0)),
            pl.BlockSpec((M_DEC, 128), lambda i: (i + N // M_DEC, 0)),
            pl.BlockSpec((M_DEC, 1), lambda i: (i, 0)),
            pl.BlockSpec((M_DEC, 1), lambda i: (i, 0)),
            full((Z_DIM, HID)),
            full((1, HID)),
            full((HID, IN_DIM)),
            full((1, IN_DIM)),
        ],
        out_specs=[
            pl.BlockSpec((M_DEC, IN_DIM), lambda i: (i, 0)),
            pl.BlockSpec((M_DEC, Z_DIM), lambda i: (i, 0)),
        ],
        out_shape=[
            jax.ShapeDtypeStruct((N, IN_DIM), F32),
            jax.ShapeDtypeStruct((N, Z_DIM), F32),
        ],
    )(zenc, gathered, gathered, idx1, idx2, W3, b3.reshape(1, HID), W4,
      b4.reshape(1, IN_DIM))

    return (recon.reshape(B, S, IN_DIM), zenc.reshape(B, S, Z_DIM),
            zemb.reshape(B, S, Z_DIM))


def kernel(X, W1, b1, W2, b2, embd, W3, b3, W4, b4):
    return _run(X, W1, b1, W2, b2, embd, W3, b3, W4, b4)
